# baseline (speedup 1.0000x reference)
import functools

import jax
import jax.numpy as jnp
from jax import lax
from jax.experimental import pallas as pl
from jax.experimental.pallas import tpu as pltpu
from jax.experimental.pallas import tpu_sc as plsc

VOCAB = 32000
EMB = 128
HID = 256
B = 32
S = 64
T = 64
G4 = 4 * HID
N_TOK = (S + T) * B
VT = 1280


def _gather_rows_sc(table, idx):
    info = plsc.get_sparse_core_info()
    nc, ns = info.num_cores, info.num_subcores
    nw = nc * ns
    per_w = N_TOK // nw
    mesh = plsc.VectorSubcoreMesh(core_axis_name="c", subcore_axis_name="s")

    @functools.partial(
        pl.kernel,
        mesh=mesh,
        out_type=jax.ShapeDtypeStruct((N_TOK, EMB), jnp.float32),
        scratch_types=[
            pltpu.VMEM((per_w,), jnp.int32),
            pltpu.VMEM((per_w, EMB), jnp.float32),
            pltpu.SemaphoreType.DMA,
        ],
    )
    def gk(table_hbm, idx_hbm, out_hbm, idx_v, rows_v, sem):
        wid = lax.axis_index("s") * nc + lax.axis_index("c")
        base = wid * per_w
        pltpu.sync_copy(idx_hbm.at[pl.ds(base, per_w)], idx_v)
        pltpu.async_copy(table_hbm.at[idx_v], rows_v, sem).wait()
        pltpu.sync_copy(rows_v, out_hbm.at[pl.ds(base, per_w)])

    return gk(table, idx)


def _nt_dot(a, b):
    return lax.dot_general(a, b, (((1,), (1,)), ((), ())),
                           preferred_element_type=jnp.float32)


def _lstm_kernel(x_ref, ewih_ref, ewhh_ref, eb_ref, dwih_ref, dwhh_ref,
                 db_ref, out_ref, xw_ref):
    xw_ref[: S * B] = _nt_dot(x_ref[: S * B], ewih_ref[...]) + eb_ref[...]
    xw_ref[S * B:] = _nt_dot(x_ref[S * B:], dwih_ref[...]) + db_ref[...]

    def cell(gates, c):
        sig = jax.nn.sigmoid(gates[:, : 3 * HID])
        i = sig[:, :HID]
        f = sig[:, HID: 2 * HID]
        o = sig[:, 2 * HID:]
        g = jnp.tanh(gates[:, 3 * HID:])
        c = f * c + i * g
        h = o * jnp.tanh(c)
        return h, c

    def enc_step(t, carry):
        h, c = carry
        gates = xw_ref[pl.ds(t * B, B)] + _nt_dot(h, ewhh_ref[...])
        return cell(gates, c)

    zeros = jnp.zeros((B, HID), jnp.float32)
    h, c = lax.fori_loop(0, S, enc_step, (zeros, zeros))

    def dec_step(t, carry):
        h, c = carry
        gates = xw_ref[pl.ds((S + t) * B, B)] + _nt_dot(h, dwhh_ref[...])
        h, c = cell(gates, c)
        out_ref[pl.ds(t * B, B)] = h
        return h, c

    lax.fori_loop(0, T, dec_step, (h, c))


def _lstm_call(x, ewih, ewhh, eb, dwih, dwhh, db):
    return pl.pallas_call(
        _lstm_kernel,
        out_shape=jax.ShapeDtypeStruct((T * B, HID), jnp.float32),
        scratch_shapes=[pltpu.VMEM((N_TOK, G4), jnp.float32)],
    )(x, ewih, ewhh, eb, dwih, dwhh, db)


def _fc_kernel(x_ref, w_ref, b_ref, o_ref):
    o_ref[...] = _nt_dot(x_ref[...], w_ref[...]) + b_ref[...]


def _fc_call(x, fc_w, fc_b2):
    return pl.pallas_call(
        _fc_kernel,
        grid=(VOCAB // VT,),
        in_specs=[
            pl.BlockSpec((B * T, HID), lambda i: (0, 0)),
            pl.BlockSpec((VT, HID), lambda i: (i, 0)),
            pl.BlockSpec((1, VT), lambda i: (0, i)),
        ],
        out_specs=pl.BlockSpec((B * T, VT), lambda i: (0, i)),
        out_shape=jax.ShapeDtypeStruct((B * T, VOCAB), jnp.float32),
    )(x, fc_w, fc_b2)


def _permute_gates(w):
    i, f, g, o = jnp.split(w, 4, axis=0)
    return jnp.concatenate([i, f, o, g], axis=0)


def kernel(src, tgt, emb, enc_W_ih, enc_W_hh, enc_b_ih, enc_b_hh,
           dec_W_ih, dec_W_hh, dec_b_ih, dec_b_hh, fc_W, fc_b):
    idx = jnp.concatenate([src.T.reshape(-1), tgt.T.reshape(-1)])
    idx = idx.astype(jnp.int32)
    x = _gather_rows_sc(emb, idx)

    ewih = _permute_gates(enc_W_ih)
    ewhh = _permute_gates(enc_W_hh)
    eb = _permute_gates(enc_b_ih + enc_b_hh)
    dwih = _permute_gates(dec_W_ih)
    dwhh = _permute_gates(dec_W_hh)
    db = _permute_gates(dec_b_ih + dec_b_hh)
    eb = eb.reshape(1, G4)
    db = db.reshape(1, G4)

    dec_hs = _lstm_call(x, ewih, ewhh, eb, dwih, dwhh, db)
    xin = dec_hs.reshape(T, B, HID).transpose(1, 0, 2).reshape(B * T, HID)
    logits = _fc_call(xin.astype(jnp.bfloat16), fc_W.astype(jnp.bfloat16),
                      fc_b.reshape(1, VOCAB))
    return logits.reshape(B, T, VOCAB)


# speedup vs baseline: 1.0782x; 1.0782x over previous
import functools

import jax
import jax.numpy as jnp
from jax import lax
from jax.experimental import pallas as pl
from jax.experimental.pallas import tpu as pltpu
from jax.experimental.pallas import tpu_sc as plsc

VOCAB = 32000
EMB = 128
HID = 256
B = 32
S = 64
T = 64
G4 = 4 * HID
N_TOK = (S + T) * B
VT = 1280


def _gather_rows_sc(table, idx):
    info = plsc.get_sparse_core_info()
    nc, ns = info.num_cores, info.num_subcores
    nw = nc * ns
    per_w = N_TOK // nw
    mesh = plsc.VectorSubcoreMesh(core_axis_name="c", subcore_axis_name="s")

    @functools.partial(
        pl.kernel,
        mesh=mesh,
        out_type=jax.ShapeDtypeStruct((N_TOK, EMB), jnp.float32),
        scratch_types=[
            pltpu.VMEM((per_w,), jnp.int32),
            pltpu.VMEM((per_w, EMB), jnp.float32),
            pltpu.SemaphoreType.DMA,
        ],
    )
    def gk(table_hbm, idx_hbm, out_hbm, idx_v, rows_v, sem):
        wid = lax.axis_index("s") * nc + lax.axis_index("c")
        base = wid * per_w
        pltpu.sync_copy(idx_hbm.at[pl.ds(base, per_w)], idx_v)
        pltpu.async_copy(table_hbm.at[idx_v], rows_v, sem).wait()
        pltpu.sync_copy(rows_v, out_hbm.at[pl.ds(base, per_w)])

    return gk(table, idx)


def _nt_dot(a, b):
    return lax.dot_general(a, b, (((1,), (1,)), ((), ())),
                           preferred_element_type=jnp.float32)


def _lstm_kernel(x_ref, ewih_ref, ewhh_ref, eb_ref, dwih_ref, dwhh_ref,
                 db_ref, out_ref, xw_ref):
    xb = x_ref[...].astype(jnp.bfloat16)
    xw_ref[: S * B] = _nt_dot(xb[: S * B], ewih_ref[...]) + eb_ref[...]
    xw_ref[S * B:] = _nt_dot(xb[S * B:], dwih_ref[...]) + db_ref[...]

    def cell(gates, c):
        sig = jax.nn.sigmoid(gates[:, : 3 * HID])
        i = sig[:, :HID]
        f = sig[:, HID: 2 * HID]
        o = sig[:, 2 * HID:]
        g = jnp.tanh(gates[:, 3 * HID:])
        c = f * c + i * g
        h = o * jnp.tanh(c)
        return h, c

    def enc_step(t, carry):
        h, c = carry
        gates = (xw_ref[pl.ds(t * B, B)]
                 + _nt_dot(h.astype(jnp.bfloat16), ewhh_ref[...]))
        return cell(gates, c)

    zeros = jnp.zeros((B, HID), jnp.float32)
    h, c = lax.fori_loop(0, S, enc_step, (zeros, zeros))

    def dec_step(t, carry):
        h, c = carry
        gates = (xw_ref[pl.ds((S + t) * B, B)]
                 + _nt_dot(h.astype(jnp.bfloat16), dwhh_ref[...]))
        h, c = cell(gates, c)
        out_ref[pl.ds(t * B, B)] = h
        return h, c

    lax.fori_loop(0, T, dec_step, (h, c))


def _lstm_call(x, ewih, ewhh, eb, dwih, dwhh, db):
    return pl.pallas_call(
        _lstm_kernel,
        out_shape=jax.ShapeDtypeStruct((T * B, HID), jnp.float32),
        scratch_shapes=[pltpu.VMEM((N_TOK, G4), jnp.float32)],
    )(x, ewih, ewhh, eb, dwih, dwhh, db)


def _fc_kernel(x_ref, w_ref, b_ref, o_ref):
    o_ref[...] = _nt_dot(x_ref[...], w_ref[...]) + b_ref[...]


def _fc_call(x, fc_w, fc_b2):
    return pl.pallas_call(
        _fc_kernel,
        grid=(VOCAB // VT,),
        in_specs=[
            pl.BlockSpec((B * T, HID), lambda i: (0, 0)),
            pl.BlockSpec((VT, HID), lambda i: (i, 0)),
            pl.BlockSpec((1, VT), lambda i: (0, i)),
        ],
        out_specs=pl.BlockSpec((B * T, VT), lambda i: (0, i)),
        out_shape=jax.ShapeDtypeStruct((B * T, VOCAB), jnp.float32),
    )(x, fc_w, fc_b2)


def _permute_gates(w):
    i, f, g, o = jnp.split(w, 4, axis=0)
    return jnp.concatenate([i, f, o, g], axis=0)


def kernel(src, tgt, emb, enc_W_ih, enc_W_hh, enc_b_ih, enc_b_hh,
           dec_W_ih, dec_W_hh, dec_b_ih, dec_b_hh, fc_W, fc_b):
    idx = jnp.concatenate([src.T.reshape(-1), tgt.T.reshape(-1)])
    idx = idx.astype(jnp.int32)
    x = _gather_rows_sc(emb, idx)

    ewih = _permute_gates(enc_W_ih).astype(jnp.bfloat16)
    ewhh = _permute_gates(enc_W_hh).astype(jnp.bfloat16)
    eb = _permute_gates(enc_b_ih + enc_b_hh)
    dwih = _permute_gates(dec_W_ih).astype(jnp.bfloat16)
    dwhh = _permute_gates(dec_W_hh).astype(jnp.bfloat16)
    db = _permute_gates(dec_b_ih + dec_b_hh)
    eb = eb.reshape(1, G4)
    db = db.reshape(1, G4)

    dec_hs = _lstm_call(x, ewih, ewhh, eb, dwih, dwhh, db)
    xin = dec_hs.reshape(T, B, HID).transpose(1, 0, 2).reshape(B * T, HID)
    logits = _fc_call(xin, fc_W, fc_b.reshape(1, VOCAB))
    return logits.reshape(B, T, VOCAB)
